# baseline (device time: 21016 ns/iter reference)
import functools

import jax
import jax.numpy as jnp
from jax import lax
from jax.experimental import pallas as pl
from jax.experimental.pallas import tpu as pltpu

N_DEV = 8
GROUP = 128
N_GROUPS = 2048 // GROUP
CHUNK_GROUPS = 4
N_CHUNKS = N_GROUPS // CHUNK_GROUPS
PRE_GROUPS = 7


def kernel(x):
    m, n = x.shape

    def body(
        x_hbm,
        out_hbm,
        xv,
        ov,
        send_row,
        totals_buf,
        in_sems,
        out_sems,
        send_sems,
        recv_sems,
    ):
        my_pos = lax.axis_index("i")
        rows_per_chunk = CHUNK_GROUPS * GROUP

        in_copies = []
        for c in range(N_CHUNKS):
            cp = pltpu.make_async_copy(
                x_hbm.at[pl.ds(c * rows_per_chunk, rows_per_chunk), :],
                xv.at[pl.ds(c * rows_per_chunk, rows_per_chunk), :],
                in_sems.at[c],
            )
            cp.start()
            in_copies.append(cp)

        barrier_sem = pltpu.get_barrier_semaphore()
        for off in range(1, N_DEV):
            pl.semaphore_signal(
                barrier_sem, inc=1,
                device_id=(lax.rem(my_pos + off, N_DEV),),
                device_id_type=pl.DeviceIdType.MESH,
            )
        pl.semaphore_wait(barrier_sem, N_DEV - 1)

        ones_row = jnp.ones((1, n), jnp.float32)

        gts = []
        for c in range(N_CHUNKS):
            in_copies[c].wait()
            for gc in range(CHUNK_GROUPS):
                g = c * CHUNK_GROUPS + gc
                u = xv[pl.ds(g * GROUP, GROUP), :]
                r = GROUP
                while r > 1:
                    u = u[: r // 2] * u[r // 2 : r]
                    r //= 2
                gts.append(u)
        gps = [ones_row]
        for g in range(1, N_GROUPS):
            gps.append(gps[g - 1] * gts[g - 1])
        send_row[...] = gps[-1] * gts[-1]

        descs = []
        for o in range(1, N_DEV):
            rdma = pltpu.make_async_remote_copy(
                src_ref=send_row,
                dst_ref=totals_buf.at[pl.ds(o, 1)],
                send_sem=send_sems.at[o],
                recv_sem=recv_sems.at[o],
                device_id=(lax.rem(my_pos + o, N_DEV),),
                device_id_type=pl.DeviceIdType.MESH,
            )
            descs.append(rdma)

            @pl.when(my_pos + o < N_DEV)
            def _():
                rdma.start()

        def scan_group(g, carry):
            v = xv[pl.ds(g * GROUP, GROUP), :]
            d = 1
            while d < GROUP:
                shifted = jnp.concatenate(
                    [jnp.ones((d, n), jnp.float32), v[: GROUP - d]], axis=0
                )
                v = v * shifted
                d *= 2
            ov[pl.ds(g * GROUP, GROUP), :] = v * carry

        out_copies = []

        def flush_group(g):
            cp = pltpu.make_async_copy(
                ov.at[pl.ds(g * GROUP, GROUP), :],
                out_hbm.at[pl.ds(g * GROUP, GROUP), :],
                out_sems.at[g],
            )
            cp.start()
            out_copies.append(cp)

        for g in range(PRE_GROUPS):
            scan_group(g, gps[g])

        for o in range(1, N_DEV):
            rdma = descs[o - 1]

            @pl.when(o <= my_pos)
            def _():
                rdma.wait_recv()

        row = lax.broadcasted_iota(jnp.int32, (N_DEV, n), 0)
        mask = (row >= 1) & (row <= my_pos)
        t = jnp.where(mask, totals_buf[...], jnp.ones((N_DEV, n), jnp.float32))
        t = t[0:4] * t[4:8]
        t = t[0:2] * t[2:4]
        pre = t[0:1] * t[1:2]

        for g in range(PRE_GROUPS, N_GROUPS):
            scan_group(g, gps[g] * pre)
            flush_group(g)

        for g in range(PRE_GROUPS):
            ov[pl.ds(g * GROUP, GROUP), :] = (
                ov[pl.ds(g * GROUP, GROUP), :] * pre
            )
            flush_group(g)

        for cp in out_copies:
            cp.wait()
        for o in range(1, N_DEV):
            rdma = descs[o - 1]

            @pl.when(my_pos + o < N_DEV)
            def _():
                rdma.wait_send()

        @functools.partial(
            pl.run_scoped, second_barrier=pltpu.SemaphoreType.REGULAR
        )
        def _(second_barrier):
            for off in range(1, N_DEV):
                pl.semaphore_signal(
                    second_barrier, inc=1,
                    device_id=(lax.rem(my_pos + off, N_DEV),),
                    device_id_type=pl.DeviceIdType.MESH,
                )
            pl.semaphore_wait(second_barrier, N_DEV - 1)

    return pl.pallas_call(
        body,
        out_shape=jax.ShapeDtypeStruct((m, n), jnp.float32),
        in_specs=[pl.BlockSpec(memory_space=pl.ANY)],
        out_specs=pl.BlockSpec(memory_space=pl.ANY),
        scratch_shapes=[
            pltpu.VMEM((m, n), jnp.float32),
            pltpu.VMEM((m, n), jnp.float32),
            pltpu.VMEM((1, n), jnp.float32),
            pltpu.VMEM((N_DEV, n), jnp.float32),
            pltpu.SemaphoreType.DMA((N_CHUNKS,)),
            pltpu.SemaphoreType.DMA((N_GROUPS,)),
            pltpu.SemaphoreType.DMA((N_DEV,)),
            pltpu.SemaphoreType.DMA((N_DEV,)),
        ],
        compiler_params=pltpu.CompilerParams(collective_id=0),
    )(x)


# device time: 19780 ns/iter; 1.0625x vs baseline; 1.0625x over previous
import functools

import jax
import jax.numpy as jnp
from jax import lax
from jax.experimental import pallas as pl
from jax.experimental.pallas import tpu as pltpu

N_DEV = 8
GROUP = 128
N_GROUPS = 2048 // GROUP
PRE_GROUPS = 6
CDT = jnp.float32


def kernel(x):
    m, n = x.shape

    def body(x_ref, out_ref, send_row, totals_buf, send_sems, recv_sems):
        my_pos = lax.axis_index("i")

        barrier_sem = pltpu.get_barrier_semaphore()
        for off in range(1, N_DEV):
            pl.semaphore_signal(
                barrier_sem, inc=1,
                device_id=(lax.rem(my_pos + off, N_DEV),),
                device_id_type=pl.DeviceIdType.MESH,
            )
        pl.semaphore_wait(barrier_sem, N_DEV - 1)

        ones_row = jnp.ones((1, n), CDT)

        gts = []
        for g in range(N_GROUPS):
            u = x_ref[pl.ds(g * GROUP, GROUP), :].astype(CDT)
            r = GROUP
            while r > 1:
                u = u[: r // 2] * u[r // 2 : r]
                r //= 2
            gts.append(u)
        gps = [ones_row]
        for g in range(1, N_GROUPS):
            gps.append(gps[g - 1] * gts[g - 1])
        send_row[...] = (gps[-1] * gts[-1]).astype(jnp.float32)

        descs = []
        for o in range(1, N_DEV):
            rdma = pltpu.make_async_remote_copy(
                src_ref=send_row,
                dst_ref=totals_buf.at[pl.ds(o, 1)],
                send_sem=send_sems.at[o],
                recv_sem=recv_sems.at[o],
                device_id=(lax.rem(my_pos + o, N_DEV),),
                device_id_type=pl.DeviceIdType.MESH,
            )
            descs.append(rdma)

            @pl.when(my_pos + o < N_DEV)
            def _():
                rdma.start()

        def scan_group(g, carry):
            v = x_ref[pl.ds(g * GROUP, GROUP), :].astype(CDT)
            d = 1
            while d < GROUP:
                shifted = jnp.concatenate(
                    [jnp.ones((d, n), CDT), v[: GROUP - d]], axis=0
                )
                v = v * shifted
                d *= 2
            out_ref[pl.ds(g * GROUP, GROUP), :] = v * carry

        for g in range(PRE_GROUPS):
            scan_group(g, gps[g])

        for o in range(1, N_DEV):
            rdma = descs[o - 1]

            @pl.when(o <= my_pos)
            def _():
                rdma.wait_recv()

        row = lax.broadcasted_iota(jnp.int32, (N_DEV, n), 0)
        mask = (row >= 1) & (row <= my_pos)
        t = jnp.where(
            mask, totals_buf[...], jnp.ones((N_DEV, n), jnp.float32)
        )
        t = t[0:4] * t[4:8]
        t = t[0:2] * t[2:4]
        pre = (t[0:1] * t[1:2]).astype(CDT)

        for g in range(PRE_GROUPS, N_GROUPS):
            scan_group(g, gps[g] * pre)

        for g in range(PRE_GROUPS):
            out_ref[pl.ds(g * GROUP, GROUP), :] = (
                out_ref[pl.ds(g * GROUP, GROUP), :] * pre
            )

        for o in range(1, N_DEV):
            rdma = descs[o - 1]

            @pl.when(my_pos + o < N_DEV)
            def _():
                rdma.wait_send()

        @functools.partial(
            pl.run_scoped, second_barrier=pltpu.SemaphoreType.REGULAR
        )
        def _(second_barrier):
            for off in range(1, N_DEV):
                pl.semaphore_signal(
                    second_barrier, inc=1,
                    device_id=(lax.rem(my_pos + off, N_DEV),),
                    device_id_type=pl.DeviceIdType.MESH,
                )
            pl.semaphore_wait(second_barrier, N_DEV - 1)

    return pl.pallas_call(
        body,
        out_shape=jax.ShapeDtypeStruct((m, n), CDT),
        in_specs=[pl.BlockSpec(memory_space=pltpu.VMEM)],
        out_specs=pl.BlockSpec(memory_space=pltpu.VMEM),
        scratch_shapes=[
            pltpu.VMEM((1, n), jnp.float32),
            pltpu.VMEM((N_DEV, n), jnp.float32),
            pltpu.SemaphoreType.DMA((N_DEV,)),
            pltpu.SemaphoreType.DMA((N_DEV,)),
        ],
        compiler_params=pltpu.CompilerParams(collective_id=0),
    )(x)


# device time: 17959 ns/iter; 1.1702x vs baseline; 1.1014x over previous
import functools

import jax
import jax.numpy as jnp
from jax import lax
from jax.experimental import pallas as pl
from jax.experimental.pallas import tpu as pltpu

N_DEV = 8
GROUP = 128
N_GROUPS = 2048 // GROUP


def kernel(x):
    m, n = x.shape

    def body(x_ref, out_ref, send_row, totals_buf, send_sems, recv_sems):
        my_pos = lax.axis_index("i")

        barrier_sem = pltpu.get_barrier_semaphore()
        for off in range(1, N_DEV):
            pl.semaphore_signal(
                barrier_sem, inc=1,
                device_id=(lax.rem(my_pos + off, N_DEV),),
                device_id_type=pl.DeviceIdType.MESH,
            )
        pl.semaphore_wait(barrier_sem, N_DEV - 1)

        ones_row = jnp.ones((1, n), jnp.float32)

        gts = []
        for g in range(N_GROUPS):
            u = x_ref[pl.ds(g * GROUP, GROUP), :]
            r = GROUP
            while r > 1:
                u = u[: r // 2] * u[r // 2 : r]
                r //= 2
            gts.append(u)
        gps = [ones_row]
        for g in range(1, N_GROUPS):
            gps.append(gps[g - 1] * gts[g - 1])
        send_row[...] = gps[-1] * gts[-1]

        descs = []
        for o in range(1, N_DEV):
            rdma = pltpu.make_async_remote_copy(
                src_ref=send_row,
                dst_ref=totals_buf.at[pl.ds(o, 1)],
                send_sem=send_sems.at[o],
                recv_sem=recv_sems.at[o],
                device_id=(lax.rem(my_pos + o, N_DEV),),
                device_id_type=pl.DeviceIdType.MESH,
            )
            descs.append(rdma)

            @pl.when(my_pos + o < N_DEV)
            def _():
                rdma.start()

        for g in range(N_GROUPS):
            v = x_ref[pl.ds(g * GROUP, GROUP), :]
            d = 1
            while d < GROUP:
                shifted = jnp.concatenate(
                    [jnp.ones((d, n), jnp.float32), v[: GROUP - d]], axis=0
                )
                v = v * shifted
                d *= 2
            out_ref[pl.ds(g * GROUP, GROUP), :] = v * gps[g]

        for o in range(1, N_DEV):
            rdma = descs[o - 1]

            @pl.when(o <= my_pos)
            def _():
                rdma.wait_recv()

        row = lax.broadcasted_iota(jnp.int32, (N_DEV, n), 0)
        mask = (row >= 1) & (row <= my_pos)
        t = jnp.where(mask, totals_buf[...], jnp.ones((N_DEV, n), jnp.float32))
        t = t[0:4] * t[4:8]
        t = t[0:2] * t[2:4]
        pre = t[0:1] * t[1:2]

        for g in range(N_GROUPS):
            out_ref[pl.ds(g * GROUP, GROUP), :] = (
                out_ref[pl.ds(g * GROUP, GROUP), :] * pre
            )

        for o in range(1, N_DEV):
            rdma = descs[o - 1]

            @pl.when(my_pos + o < N_DEV)
            def _():
                rdma.wait_send()

        @functools.partial(
            pl.run_scoped, second_barrier=pltpu.SemaphoreType.REGULAR
        )
        def _(second_barrier):
            for off in range(1, N_DEV):
                pl.semaphore_signal(
                    second_barrier, inc=1,
                    device_id=(lax.rem(my_pos + off, N_DEV),),
                    device_id_type=pl.DeviceIdType.MESH,
                )
            pl.semaphore_wait(second_barrier, N_DEV - 1)

    return pl.pallas_call(
        body,
        out_shape=jax.ShapeDtypeStruct((m, n), jnp.float32),
        in_specs=[pl.BlockSpec(memory_space=pltpu.VMEM)],
        out_specs=pl.BlockSpec(memory_space=pltpu.VMEM),
        scratch_shapes=[
            pltpu.VMEM((1, n), jnp.float32),
            pltpu.VMEM((N_DEV, n), jnp.float32),
            pltpu.SemaphoreType.DMA((N_DEV,)),
            pltpu.SemaphoreType.DMA((N_DEV,)),
        ],
        compiler_params=pltpu.CompilerParams(collective_id=0),
    )(x)
